# baseline (speedup 1.0000x reference)
import jax
import jax.numpy as jnp
from jax.experimental import pallas as pl

N = 10000
E = 5000
NNZ = 320000
D = 128
LEVELS = (0, 1, 2, 4, 8, 16)


def _wavelet_body(levels_ref, out_ref):
    for w in range(6):
        cur = levels_ref[w]
        nxt = levels_ref[w + 1] if w < 5 else jnp.zeros_like(cur)
        coeff = cur - nxt
        pos = jnp.maximum(coeff, 0.0)
        neg = jnp.maximum(-coeff, 0.0)
        inter = jnp.stack([pos, neg], axis=-1).reshape(cur.shape[0], D * 2)
        out_ref[:, w * D * 2:(w + 1) * D * 2] = inter


def _wavelet_pallas(levels, rows):
    BLK = 40
    grid = rows // BLK
    return pl.pallas_call(
        _wavelet_body,
        grid=(grid,),
        in_specs=[pl.BlockSpec((6, BLK, D), lambda i: (0, i, 0))],
        out_specs=pl.BlockSpec((BLK, 6 * D * 2), lambda i: (i, 0)),
        out_shape=jax.ShapeDtypeStruct((rows, 6 * D * 2), jnp.float32),
    )(levels)


def kernel(X, Y, incidence_v, incidence_e, W):
    iv = incidence_v.astype(jnp.int32)
    ie = incidence_e.astype(jnp.int32)
    ones = jnp.ones((NNZ,), dtype=jnp.float32)
    deg_v = jax.ops.segment_sum(ones, iv, num_segments=N)
    deg_e = jax.ops.segment_sum(ones, ie, num_segments=E)
    inv_deg_v = jnp.where(deg_v > 0, 1.0 / deg_v, 0.0)
    inv_deg_e = jnp.where(deg_e > 0, 1.0 / deg_e, 0.0)

    node_keep = {0: X}
    edge_keep = {0: Y}
    Xc = X
    for t in range(1, 17):
        X_norm = Xc * inv_deg_v[:, None]
        edge_feat = jax.ops.segment_sum(X_norm[iv], ie, num_segments=E)
        edge_feat_norm = edge_feat * inv_deg_e[:, None]
        Xc = jax.ops.segment_sum(edge_feat_norm[ie], iv, num_segments=N)
        if t in LEVELS:
            node_keep[t] = Xc
            edge_keep[t] = edge_feat

    node_levels = jnp.stack([node_keep[t] for t in LEVELS], axis=0)
    edge_levels = jnp.stack([edge_keep[t] for t in LEVELS], axis=0)
    s_nodes = _wavelet_pallas(node_levels, N)
    s_edges = _wavelet_pallas(edge_levels, E)
    return (s_nodes, s_edges)


# speedup vs baseline: 4.1553x; 4.1553x over previous
import functools

import jax
import jax.numpy as jnp
from jax import lax
from jax.experimental import pallas as pl
from jax.experimental.pallas import tpu as pltpu
from jax.experimental.pallas import tpu_sc as plsc

N = 10000
E = 5000
NNZ = 320000
D = 128
KEPT = (1, 2, 4, 8, 16)
NP = 10112
EP = 5120

_mesh = plsc.VectorSubcoreMesh(core_axis_name="c", subcore_axis_name="s")
_f32 = jnp.float32


def _zero_rows128(buf, nrows):
    def body(r, _):
        for cidx in range(8):
            buf[r, pl.ds(cidx * 16, 16)] = jnp.zeros((16,), _f32)
        return 0
    lax.fori_loop(0, nrows, body, 0)


def _make_scatter(dst_pad):
    PPT = NNZ // 32
    C = 200
    NCH = PPT // C
    ZR = dst_pad // 16

    @functools.partial(
        pl.kernel,
        out_type=jax.ShapeDtypeStruct((2, dst_pad, 128), _f32),
        mesh=_mesh,
        scratch_types=[
            pltpu.VMEM((C, 128), _f32),
            pltpu.VMEM((C,), jnp.int32),
            pltpu.VMEM((C,), jnp.int32),
            pltpu.VMEM_SHARED((dst_pad, 128), _f32),
            pltpu.SemaphoreType.DMA,
        ],
    )
    def scat(src, gidx, sidx, out, rows, gib, sib, acc, sem):
        c = lax.axis_index("c")
        s = lax.axis_index("s")
        wid = c * 16 + s
        _zero_rows128(rows, C)
        done = 0
        while done < ZR:
            step = min(C, ZR - done)
            o = pl.multiple_of(s * ZR + done, 8)
            pltpu.sync_copy(rows.at[pl.ds(0, step)], acc.at[pl.ds(o, step)])
            done += step
        plsc.subcore_barrier()

        def chunk(ch, _):
            b = pl.multiple_of(wid * PPT + ch * C, 8)
            pltpu.sync_copy(gidx.at[pl.ds(b, C)], gib)
            pltpu.sync_copy(sidx.at[pl.ds(b, C)], sib)
            pltpu.async_copy(src.at[gib], rows, sem).wait()
            pltpu.sync_copy(rows, acc.at[sib], add=True)
            return 0
        lax.fori_loop(0, NCH, chunk, 0)
        plsc.subcore_barrier()
        o = pl.multiple_of(s * ZR, 8)
        pltpu.sync_copy(acc.at[pl.ds(o, ZR)], out.at[c, pl.ds(o, ZR)])

    return scat


_scatter_e = _make_scatter(EP)
_scatter_n = _make_scatter(NP)


def _inv_body(dp_ref, out_ref):
    d = dp_ref[0, :, :16] + dp_ref[1, :, :16]
    out_ref[...] = jnp.where(d > 0, 1.0 / d, 0.0)


def _inv_deg(dp, rows):
    blk = 1000
    return pl.pallas_call(
        _inv_body,
        grid=(rows // blk,),
        in_specs=[pl.BlockSpec((2, blk, 128), lambda i: (0, i, 0))],
        out_specs=pl.BlockSpec((blk, 16), lambda i: (i, 0)),
        out_shape=jax.ShapeDtypeStruct((rows, 16), _f32),
    )(dp)


def _norm0_body(x_ref, inv_ref, out_ref):
    out_ref[...] = x_ref[...] * inv_ref[:, :1]


def _norm0(x, inv, rows):
    blk = 1000
    return pl.pallas_call(
        _norm0_body,
        grid=(rows // blk,),
        in_specs=[pl.BlockSpec((blk, 128), lambda i: (i, 0)),
                  pl.BlockSpec((blk, 16), lambda i: (i, 0))],
        out_specs=pl.BlockSpec((blk, 128), lambda i: (i, 0)),
        out_shape=jax.ShapeDtypeStruct((rows, 128), _f32),
    )(x, inv)


def _comb_body(p_ref, inv_ref, out_ref):
    ssum = p_ref[0] + p_ref[1]
    out_ref[...] = ssum * inv_ref[:, :1]


def _comb_lvl_body(p_ref, inv_ref, out_ref, lvl_ref):
    ssum = p_ref[0] + p_ref[1]
    lvl_ref[...] = ssum
    out_ref[...] = ssum * inv_ref[:, :1]


def _combine(partial, inv, rows, want_level):
    blk = 1000
    body = _comb_lvl_body if want_level else _comb_body
    out_shape = jax.ShapeDtypeStruct((rows, 128), _f32)
    if want_level:
        out_shape = (out_shape, jax.ShapeDtypeStruct((rows, 128), _f32))
        out_specs = (pl.BlockSpec((blk, 128), lambda i: (i, 0)),
                     pl.BlockSpec((blk, 128), lambda i: (i, 0)))
    else:
        out_specs = pl.BlockSpec((blk, 128), lambda i: (i, 0))
    return pl.pallas_call(
        body,
        grid=(rows // blk,),
        in_specs=[pl.BlockSpec((2, blk, 128), lambda i: (0, i, 0)),
                  pl.BlockSpec((blk, 16), lambda i: (i, 0))],
        out_specs=out_specs,
        out_shape=out_shape,
    )(partial, inv)


def _wavelet_body(l0, l1, l2, l3, l4, l5, out_ref):
    refs = (l0, l1, l2, l3, l4, l5)
    for w in range(6):
        cur = refs[w][...]
        coeff = cur - refs[w + 1][...] if w < 5 else cur
        pos = jnp.maximum(coeff, 0.0)
        neg = jnp.maximum(-coeff, 0.0)
        inter = jnp.stack([pos, neg], axis=-1).reshape(cur.shape[0], D * 2)
        out_ref[:, w * D * 2:(w + 1) * D * 2] = inter


def _wavelet(levels, rows):
    BLK = 40
    spec = pl.BlockSpec((BLK, D), lambda i: (i, 0))
    return pl.pallas_call(
        _wavelet_body,
        grid=(rows // BLK,),
        in_specs=[spec] * 6,
        out_specs=pl.BlockSpec((BLK, 6 * D * 2), lambda i: (i, 0)),
        out_shape=jax.ShapeDtypeStruct((rows, 6 * D * 2), _f32),
    )(*levels)


def kernel(X, Y, incidence_v, incidence_e, W):
    iv = incidence_v.astype(jnp.int32)
    ie = incidence_e.astype(jnp.int32)

    de_p = _scatter_e(jnp.ones((N, 128), _f32), iv, ie)
    dv_p = _scatter_n(jnp.ones((E, 128), _f32), ie, iv)
    inv_v = _inv_deg(dv_p, N)
    inv_e = _inv_deg(de_p, E)

    xn = _norm0(X, inv_v, N)
    node_lvls = {0: X}
    edge_lvls = {0: Y}
    for t in range(1, 17):
        e_part = _scatter_e(xn, iv, ie)
        if t in KEPT:
            en, elvl = _combine(e_part, inv_e, E, True)
            edge_lvls[t] = elvl
        else:
            en = _combine(e_part, inv_e, E, False)
        n_part = _scatter_n(en, ie, iv)
        if t in KEPT:
            xn, nlvl = _combine(n_part, inv_v, N, True)
            node_lvls[t] = nlvl
        else:
            xn = _combine(n_part, inv_v, N, False)

    kept = (0,) + KEPT
    s_nodes = _wavelet([node_lvls[t] for t in kept], N)
    s_edges = _wavelet([edge_lvls[t] for t in kept], E)
    return (s_nodes, s_edges)
